# baseline (speedup 1.0000x reference)
import functools

import jax
import jax.numpy as jnp
from jax import lax
from jax.experimental import pallas as pl
from jax.experimental.pallas import tpu as pltpu
from jax.experimental.pallas import tpu_sc as plsc

N_ASPECTS = 5
BATCH = 16384
NUM_CORES = 2
NUM_SUBCORES = 16
LANES = 16
NW = NUM_CORES * NUM_SUBCORES
BPW = BATCH // NW
CHUNKS = BPW // LANES

_mesh = plsc.VectorSubcoreMesh(
    core_axis_name="c", subcore_axis_name="s",
    num_cores=NUM_CORES, num_subcores=NUM_SUBCORES)


@functools.partial(
    pl.kernel,
    out_type=jax.ShapeDtypeStruct((BATCH,), jnp.float32),
    mesh=_mesh,
    scratch_types=[
        pltpu.VMEM((BPW,), jnp.int32),
        *[pltpu.VMEM((BPW,), jnp.int32) for _ in range(N_ASPECTS)],
        *[pltpu.VMEM((BPW,), jnp.float32) for _ in range(N_ASPECTS)],
        *[pltpu.VMEM((BPW,), jnp.float32) for _ in range(N_ASPECTS)],
        pltpu.VMEM((BPW,), jnp.float32),
        pltpu.SemaphoreType.DMA,
    ],
)
def _sc_kernel(ids_hbm, ratings_t_hbm, table_hbm, out_hbm,
               ids_v, *rest):
    idx = rest[:N_ASPECTS]
    wc = rest[N_ASPECTS:2 * N_ASPECTS]
    rc = rest[2 * N_ASPECTS:3 * N_ASPECTS]
    o_v, sem = rest[-2], rest[-1]
    wid = lax.axis_index("s") * NUM_CORES + lax.axis_index("c")
    base = wid * BPW
    pltpu.sync_copy(ids_hbm.at[pl.ds(base, BPW)], ids_v)
    rcopies = [
        pltpu.async_copy(
            ratings_t_hbm.at[pl.ds(a * BATCH + base, BPW)], rc[a], sem)
        for a in range(N_ASPECTS)]

    def idx_body(c, _):
        sl = pl.ds(c * LANES, LANES)
        flat = ids_v[sl] * N_ASPECTS
        for a in range(N_ASPECTS):
            idx[a][sl] = flat + a
        return _

    lax.fori_loop(0, CHUNKS, idx_body, None)

    gathers = [pltpu.async_copy(table_hbm.at[idx[a]], wc[a], sem)
               for a in range(N_ASPECTS)]
    for c in rcopies:
        c.wait()
    for g in gathers:
        g.wait()

    def body(c, _):
        sl = pl.ds(c * LANES, LANES)
        s = jnp.zeros((LANES,), jnp.float32)
        dot = jnp.zeros((LANES,), jnp.float32)
        for a in range(N_ASPECTS):
            w = wc[a][sl]
            r = rc[a][sl]
            s = s + jnp.abs(w)
            dot = dot + w * r
        o_v[sl] = dot / jnp.maximum(s, 1e-12)
        return _

    lax.fori_loop(0, CHUNKS, body, None)
    pltpu.sync_copy(o_v, out_hbm.at[pl.ds(base, BPW)])


def kernel(U_ids, A_ratings, users_parameters):
    return _sc_kernel(U_ids, A_ratings.T.reshape(-1),
                      users_parameters.reshape(-1))


# speedup vs baseline: 1.7594x; 1.7594x over previous
import functools

import jax
import jax.numpy as jnp
from jax import lax
from jax.experimental import pallas as pl
from jax.experimental.pallas import tpu as pltpu
from jax.experimental.pallas import tpu_sc as plsc

N_ASPECTS = 5
BATCH = 16384
NUM_CORES = 2
NUM_SUBCORES = 16
LANES = 16
NW = NUM_CORES * NUM_SUBCORES
BPW = BATCH // NW
CHUNKS = BPW // LANES

_mesh = plsc.VectorSubcoreMesh(
    core_axis_name="c", subcore_axis_name="s",
    num_cores=NUM_CORES, num_subcores=NUM_SUBCORES)


def _worker_base():
    wid = lax.axis_index("s") * NUM_CORES + lax.axis_index("c")
    return wid * BPW


_GROUP = 16
_NGROUPS = BPW // _GROUP


@functools.partial(
    pl.kernel,
    out_type=jax.ShapeDtypeStruct((BATCH, N_ASPECTS), jnp.float32),
    mesh=_mesh,
    scratch_types=[
        pltpu.VMEM((BPW,), jnp.int32),
        pltpu.VMEM((BPW, N_ASPECTS), jnp.float32),
        pltpu.SemaphoreType.DMA,
        pltpu.SemaphoreType.DMA,
    ],
)
def _sc_gather_rows(ids_hbm, table_hbm, rows_hbm, ids_v, w_v, sem, gsem):
    base = _worker_base()
    pltpu.async_copy(ids_hbm.at[pl.ds(base, BPW)], ids_v, sem).wait()

    def enqueue_group(g):
        vec = ids_v[pl.ds(g * _GROUP, _GROUP)]
        for j in range(_GROUP):
            slot = g * _GROUP + j
            row = vec[j]
            pltpu.async_copy(table_hbm.at[pl.ds(row, 1), :],
                             w_v.at[pl.ds(slot, 1), :], gsem)

    def drain_group():
        for _ in range(_GROUP):
            pltpu.make_async_copy(table_hbm.at[pl.ds(0, 1), :],
                                  w_v.at[pl.ds(0, 1), :], gsem).wait()

    def body(g, _):
        enqueue_group(g)
        drain_group()
        return _

    lax.fori_loop(0, _NGROUPS, body, None)
    pltpu.sync_copy(w_v, rows_hbm.at[pl.ds(base, BPW)])


@functools.partial(
    pl.kernel,
    out_type=jax.ShapeDtypeStruct((BATCH,), jnp.float32),
    mesh=_mesh,
    scratch_types=[
        *[pltpu.VMEM((BPW,), jnp.int32) for _ in range(N_ASPECTS)],
        *[pltpu.VMEM((BPW,), jnp.float32) for _ in range(N_ASPECTS)],
        *[pltpu.VMEM((BPW,), jnp.float32) for _ in range(N_ASPECTS)],
        pltpu.VMEM((BPW,), jnp.float32),
        pltpu.SemaphoreType.DMA,
    ],
)
def _sc_combine(rows_flat_hbm, ratings_t_hbm, out_hbm, *rest):
    idx = rest[:N_ASPECTS]
    wc = rest[N_ASPECTS:2 * N_ASPECTS]
    rc = rest[2 * N_ASPECTS:3 * N_ASPECTS]
    o_v, sem = rest[-2], rest[-1]
    base = _worker_base()
    rcopies = [
        pltpu.async_copy(
            ratings_t_hbm.at[pl.ds(a * BATCH + base, BPW)], rc[a], sem)
        for a in range(N_ASPECTS)]

    iota = lax.iota(jnp.int32, LANES)

    def idx_body(c, _):
        sl = pl.ds(c * LANES, LANES)
        flat = (base + c * LANES + iota) * N_ASPECTS
        for a in range(N_ASPECTS):
            idx[a][sl] = flat + a
        return _

    lax.fori_loop(0, CHUNKS, idx_body, None)

    gathers = [pltpu.async_copy(rows_flat_hbm.at[idx[a]], wc[a], sem)
               for a in range(N_ASPECTS)]
    for c in rcopies:
        c.wait()
    for g in gathers:
        g.wait()

    def body(c, _):
        sl = pl.ds(c * LANES, LANES)
        s = jnp.zeros((LANES,), jnp.float32)
        dot = jnp.zeros((LANES,), jnp.float32)
        for a in range(N_ASPECTS):
            w = wc[a][sl]
            r = rc[a][sl]
            s = s + jnp.abs(w)
            dot = dot + w * r
        o_v[sl] = dot / jnp.maximum(s, 1e-12)
        return _

    lax.fori_loop(0, CHUNKS, body, None)
    pltpu.sync_copy(o_v, out_hbm.at[pl.ds(base, BPW)])


def kernel(U_ids, A_ratings, users_parameters):
    rows = _sc_gather_rows(U_ids, users_parameters)
    return _sc_combine(rows.reshape(-1), A_ratings.T.reshape(-1))
